# baseline (speedup 1.0000x reference)
import functools

import numpy as np
import jax
import jax.numpy as jnp
from jax import lax
from jax.experimental import pallas as pl
from jax.experimental.pallas import tpu as pltpu
from jax.experimental.pallas import tpu_sc as plsc

B = 4096
N = 1000
BB = 256


def _np_threefry2x32(k1, k2, x0, x1):
    k1, k2 = np.uint32(k1), np.uint32(k2)
    ks = [k1, k2, np.uint32(k1 ^ k2 ^ np.uint32(0x1BD11BDA))]
    rots = [[13, 15, 26, 6], [17, 29, 16, 24]]
    x0 = (x0 + ks[0]).astype(np.uint32)
    x1 = (x1 + ks[1]).astype(np.uint32)
    for i in range(5):
        for r in rots[i % 2]:
            x0 = (x0 + x1).astype(np.uint32)
            x1 = ((x1 << np.uint32(r)) | (x1 >> np.uint32(32 - r))).astype(np.uint32)
            x1 = (x0 ^ x1).astype(np.uint32)
        x0 = (x0 + ks[(i + 1) % 3]).astype(np.uint32)
        x1 = (x1 + ks[(i + 2) % 3] + np.uint32(i + 1)).astype(np.uint32)
    return x0, x1


def _subkeys_of_42():
    idx = np.arange(3, dtype=np.uint64)
    hi = (idx >> np.uint64(32)).astype(np.uint32)
    lo = (idx & np.uint64(0xFFFFFFFF)).astype(np.uint32)
    o0, o1 = _np_threefry2x32(np.uint32(0), np.uint32(42), hi, lo)
    return [(int(o0[i]), int(o1[i])) for i in range(3)]


_KB, _KC, _KE = _subkeys_of_42()

_TINY = np.float32(np.finfo(np.float32).tiny)
_LOG_1E9 = np.float32(np.log(np.float32(1e-9)))
_ROTS = (13, 15, 26, 6, 17, 29, 16, 24, 13, 15, 26, 6, 17, 29, 16, 24, 13, 15, 26, 6)


def _as_i32(x):
    return np.array(x, dtype=np.uint32).view(np.int32)[()]


def _rotl(x, r):
    return lax.shift_left(x, np.int32(r)) | lax.shift_right_logical(x, np.int32(32 - r))


def _key_consts(key):
    k1, k2 = np.uint32(key[0]), np.uint32(key[1])
    k3 = np.uint32(k1 ^ k2 ^ np.uint32(0x1BD11BDA))
    return [_as_i32(k1), _as_i32(k2), _as_i32(k3)]


def _threefry_bits(key, idx):
    kseq = _key_consts(key)
    x0 = jnp.full(idx.shape, kseq[0], dtype=jnp.int32)
    x1 = idx + kseq[1]
    for i in range(5):
        for r in _ROTS[i * 4 : i * 4 + 4]:
            x0 = x0 + x1
            x1 = _rotl(x1, r)
            x1 = x0 ^ x1
        x0 = x0 + kseq[(i + 1) % 3]
        x1 = x1 + kseq[(i + 2) % 3] + np.int32(i + 1)
    return x0 ^ x1


def _gumbel_from_bits(bits):
    fb = lax.shift_right_logical(bits, np.int32(9)) | np.int32(0x3F800000)
    fl = lax.bitcast_convert_type(fb, jnp.float32) - np.float32(1.0)
    u = fl + _TINY
    return -jnp.log(-jnp.log(u))


_NW = 32
_ELEMS = B * N
_PER_W = _ELEMS // _NW
_CHUNK = 32000
_VECS = _CHUNK // 16


def _sc_bits_body(out_hbm, buf):
    wid = lax.axis_index("s") * 2 + lax.axis_index("c")
    kseq = _key_consts(_KE)
    lane = lax.iota(jnp.int32, 16)
    base = wid * np.int32(_PER_W)
    for ch in range(_PER_W // _CHUNK):
        chunk_base = base + np.int32(ch * _CHUNK)

        def body(i, carry, chunk_base=chunk_base):
            idx = lane + (chunk_base + i * np.int32(16))
            buf[pl.ds(i * 16, 16)] = _threefry_bits(_KE, idx)
            return carry

        lax.fori_loop(0, _VECS, body, np.int32(0))
        pltpu.sync_copy(buf, out_hbm.at[pl.ds(chunk_base, _CHUNK)])


def _sc_bits_e():
    mesh = plsc.VectorSubcoreMesh(core_axis_name="c", subcore_axis_name="s")
    fn = pl.kernel(
        _sc_bits_body,
        mesh=mesh,
        out_type=jax.ShapeDtypeStruct((_ELEMS,), jnp.int32),
        scratch_types=[pltpu.VMEM((_CHUNK,), jnp.int32)],
    )
    return fn()


def _first_argmax(s, col):
    m = jnp.max(s, axis=1, keepdims=True)
    return jnp.min(jnp.where(s == m, col, np.int32(N)), axis=1)


def _body(pot_ref, pc_ref, pe_ref, tour_ref, be_ref, bt_ref, city_ref, es_ref, ee_ref):
    i = pl.program_id(0)
    row = lax.broadcasted_iota(jnp.int32, (BB, N), 0)
    col = lax.broadcasted_iota(jnp.int32, (BB, N), 1)
    idx = (i * np.int32(BB) + row) * np.int32(N) + col

    g_b = _gumbel_from_bits(_threefry_bits(_KB, idx))
    bt_ref[...] = _first_argmax(pot_ref[...] + g_b, col)

    g_c = _gumbel_from_bits(_threefry_bits(_KC, idx))
    city = _first_argmax(jnp.log(pc_ref[...]) + g_c, col)
    city_ref[...] = city

    g_e = _gumbel_from_bits(be_ref[...])
    s_e = jnp.where(col == city[:, None], _LOG_1E9, jnp.log(pe_ref[...])) + g_e
    ie = _first_argmax(s_e, col)
    es_ref[...] = ie

    tour = tour_ref[...]
    pos = jnp.min(jnp.where(tour == ie[:, None], col, np.int32(N)), axis=1)
    nxt = jnp.where(pos == np.int32(N - 1), np.int32(0), pos + np.int32(1))
    ee_ref[...] = jnp.sum(jnp.where(col == nxt[:, None], tour, np.int32(0)), axis=1)


def kernel(backtrack_potentials, city_to_insert_probs, edge_to_insert_probs, current_tour):
    bits_e = _sc_bits_e().reshape(B, N)
    in_spec = pl.BlockSpec((BB, N), lambda i: (i, 0))
    out_spec = pl.BlockSpec((BB,), lambda i: (i,))
    out_shape = jax.ShapeDtypeStruct((B,), jnp.int32)
    bt, city, es, ee = pl.pallas_call(
        _body,
        grid=(B // BB,),
        in_specs=[in_spec] * 5,
        out_specs=[out_spec] * 4,
        out_shape=[out_shape] * 4,
    )(backtrack_potentials, city_to_insert_probs, edge_to_insert_probs, current_tour, bits_e)
    return bt, city, jnp.stack([es, ee], axis=1)


# speedup vs baseline: 1.3856x; 1.3856x over previous
import functools

import numpy as np
import jax
import jax.numpy as jnp
from jax import lax
from jax.experimental import pallas as pl
from jax.experimental.pallas import tpu as pltpu
from jax.experimental.pallas import tpu_sc as plsc

B = 4096
N = 1000
BB = 256


def _np_threefry2x32(k1, k2, x0, x1):
    k1, k2 = np.uint32(k1), np.uint32(k2)
    ks = [k1, k2, np.uint32(k1 ^ k2 ^ np.uint32(0x1BD11BDA))]
    rots = [[13, 15, 26, 6], [17, 29, 16, 24]]
    x0 = (x0 + ks[0]).astype(np.uint32)
    x1 = (x1 + ks[1]).astype(np.uint32)
    for i in range(5):
        for r in rots[i % 2]:
            x0 = (x0 + x1).astype(np.uint32)
            x1 = ((x1 << np.uint32(r)) | (x1 >> np.uint32(32 - r))).astype(np.uint32)
            x1 = (x0 ^ x1).astype(np.uint32)
        x0 = (x0 + ks[(i + 1) % 3]).astype(np.uint32)
        x1 = (x1 + ks[(i + 2) % 3] + np.uint32(i + 1)).astype(np.uint32)
    return x0, x1


def _subkeys_of_42():
    idx = np.arange(3, dtype=np.uint64)
    hi = (idx >> np.uint64(32)).astype(np.uint32)
    lo = (idx & np.uint64(0xFFFFFFFF)).astype(np.uint32)
    o0, o1 = _np_threefry2x32(np.uint32(0), np.uint32(42), hi, lo)
    return [(int(o0[i]), int(o1[i])) for i in range(3)]


_KB, _KC, _KE = _subkeys_of_42()

_TINY = np.float32(np.finfo(np.float32).tiny)
_LOG_1E9 = np.float32(np.log(np.float32(1e-9)))
_ROTS = (13, 15, 26, 6, 17, 29, 16, 24, 13, 15, 26, 6, 17, 29, 16, 24, 13, 15, 26, 6)


def _as_i32(x):
    return np.array(x, dtype=np.uint32).view(np.int32)[()]


def _rotl(x, r):
    return lax.shift_left(x, np.int32(r)) | lax.shift_right_logical(x, np.int32(32 - r))


def _key_consts(key):
    k1, k2 = np.uint32(key[0]), np.uint32(key[1])
    k3 = np.uint32(k1 ^ k2 ^ np.uint32(0x1BD11BDA))
    return [_as_i32(k1), _as_i32(k2), _as_i32(k3)]


def _threefry_bits(key, idx):
    kseq = _key_consts(key)
    x0 = jnp.full(idx.shape, kseq[0], dtype=jnp.int32)
    x1 = idx + kseq[1]
    for i in range(5):
        for r in _ROTS[i * 4 : i * 4 + 4]:
            x0 = x0 + x1
            x1 = _rotl(x1, r)
            x1 = x0 ^ x1
        x0 = x0 + kseq[(i + 1) % 3]
        x1 = x1 + kseq[(i + 2) % 3] + np.int32(i + 1)
    return x0 ^ x1


def _gumbel_from_bits(bits):
    fb = lax.shift_right_logical(bits, np.int32(9)) | np.int32(0x3F800000)
    fl = lax.bitcast_convert_type(fb, jnp.float32) - np.float32(1.0)
    u = fl + _TINY
    return -jnp.log(-jnp.log(u))


_NW = 32
_ELEMS = B * N
_PER_W = _ELEMS // _NW
_CHUNK = 32000
_VECS = _CHUNK // 16


def _sc_bits_body(out_hbm, buf):
    wid = lax.axis_index("s") * 2 + lax.axis_index("c")
    kseq = _key_consts(_KE)
    lane = lax.iota(jnp.int32, 16)
    base = wid * np.int32(_PER_W)
    for ch in range(_PER_W // _CHUNK):
        chunk_base = base + np.int32(ch * _CHUNK)

        def body(i, carry, chunk_base=chunk_base):
            idx = lane + (chunk_base + i * np.int32(16))
            buf[pl.ds(i * 16, 16)] = _threefry_bits(_KE, idx)
            return carry

        lax.fori_loop(0, _VECS, body, np.int32(0))
        pltpu.sync_copy(buf, out_hbm.at[pl.ds(chunk_base, _CHUNK)])


def _sc_bits_e():
    mesh = plsc.VectorSubcoreMesh(core_axis_name="c", subcore_axis_name="s")
    fn = pl.kernel(
        _sc_bits_body,
        mesh=mesh,
        out_type=jax.ShapeDtypeStruct((_ELEMS,), jnp.int32),
        scratch_types=[pltpu.VMEM((_CHUNK,), jnp.int32)],
    )
    return fn()


def _first_argmax(s, col):
    m = jnp.max(s, axis=1, keepdims=True)
    return jnp.min(jnp.where(s == m, col, np.int32(N)), axis=1)


def _body1(pot_ref, pc_ref, bt_ref, city_ref):
    i = pl.program_id(0)
    row = lax.broadcasted_iota(jnp.int32, (BB, N), 0)
    col = lax.broadcasted_iota(jnp.int32, (BB, N), 1)
    idx = (i * np.int32(BB) + row) * np.int32(N) + col

    g_b = _gumbel_from_bits(_threefry_bits(_KB, idx))
    bt_ref[...] = _first_argmax(pot_ref[...] + g_b, col)

    g_c = _gumbel_from_bits(_threefry_bits(_KC, idx))
    city_ref[...] = _first_argmax(jnp.log(pc_ref[...]) + g_c, col)


def _body2(pe_ref, tour_ref, be_ref, city_ref, es_ref, ee_ref):
    col = lax.broadcasted_iota(jnp.int32, (BB, N), 1)
    city = city_ref[...]

    g_e = _gumbel_from_bits(be_ref[...])
    s_e = jnp.where(col == city[:, None], _LOG_1E9, jnp.log(pe_ref[...])) + g_e
    ie = _first_argmax(s_e, col)
    es_ref[...] = ie

    tour = tour_ref[...]
    pos = jnp.min(jnp.where(tour == ie[:, None], col, np.int32(N)), axis=1)
    nxt = jnp.where(pos == np.int32(N - 1), np.int32(0), pos + np.int32(1))
    ee_ref[...] = jnp.sum(jnp.where(col == nxt[:, None], tour, np.int32(0)), axis=1)


def kernel(backtrack_potentials, city_to_insert_probs, edge_to_insert_probs, current_tour):
    bits_e = _sc_bits_e().reshape(B, N)
    in_spec = pl.BlockSpec((BB, N), lambda i: (i, 0))
    out_spec = pl.BlockSpec((BB,), lambda i: (i,))
    out_shape = jax.ShapeDtypeStruct((B,), jnp.int32)
    bt, city = pl.pallas_call(
        _body1,
        grid=(B // BB,),
        in_specs=[in_spec] * 2,
        out_specs=[out_spec] * 2,
        out_shape=[out_shape] * 2,
    )(backtrack_potentials, city_to_insert_probs)
    es, ee = pl.pallas_call(
        _body2,
        grid=(B // BB,),
        in_specs=[in_spec, in_spec, in_spec, out_spec],
        out_specs=[out_spec] * 2,
        out_shape=[out_shape] * 2,
    )(edge_to_insert_probs, current_tour, bits_e, city)
    return bt, city, jnp.stack([es, ee], axis=1)


# speedup vs baseline: 1.3864x; 1.0005x over previous
import functools

import numpy as np
import jax
import jax.numpy as jnp
from jax import lax
from jax.experimental import pallas as pl
from jax.experimental.pallas import tpu as pltpu
from jax.experimental.pallas import tpu_sc as plsc

B = 4096
N = 1000
BB = 256


def _np_threefry2x32(k1, k2, x0, x1):
    k1, k2 = np.uint32(k1), np.uint32(k2)
    ks = [k1, k2, np.uint32(k1 ^ k2 ^ np.uint32(0x1BD11BDA))]
    rots = [[13, 15, 26, 6], [17, 29, 16, 24]]
    x0 = (x0 + ks[0]).astype(np.uint32)
    x1 = (x1 + ks[1]).astype(np.uint32)
    for i in range(5):
        for r in rots[i % 2]:
            x0 = (x0 + x1).astype(np.uint32)
            x1 = ((x1 << np.uint32(r)) | (x1 >> np.uint32(32 - r))).astype(np.uint32)
            x1 = (x0 ^ x1).astype(np.uint32)
        x0 = (x0 + ks[(i + 1) % 3]).astype(np.uint32)
        x1 = (x1 + ks[(i + 2) % 3] + np.uint32(i + 1)).astype(np.uint32)
    return x0, x1


def _subkeys_of_42():
    idx = np.arange(3, dtype=np.uint64)
    hi = (idx >> np.uint64(32)).astype(np.uint32)
    lo = (idx & np.uint64(0xFFFFFFFF)).astype(np.uint32)
    o0, o1 = _np_threefry2x32(np.uint32(0), np.uint32(42), hi, lo)
    return [(int(o0[i]), int(o1[i])) for i in range(3)]


_KB, _KC, _KE = _subkeys_of_42()

_TINY = np.float32(np.finfo(np.float32).tiny)
_LOG_1E9 = np.float32(np.log(np.float32(1e-9)))
_ROTS = (13, 15, 26, 6, 17, 29, 16, 24, 13, 15, 26, 6, 17, 29, 16, 24, 13, 15, 26, 6)


def _as_i32(x):
    return np.array(x, dtype=np.uint32).view(np.int32)[()]


def _rotl(x, r):
    return lax.shift_left(x, np.int32(r)) | lax.shift_right_logical(x, np.int32(32 - r))


def _key_consts(key):
    k1, k2 = np.uint32(key[0]), np.uint32(key[1])
    k3 = np.uint32(k1 ^ k2 ^ np.uint32(0x1BD11BDA))
    return [_as_i32(k1), _as_i32(k2), _as_i32(k3)]


def _threefry_bits(key, idx):
    kseq = _key_consts(key)
    x0 = jnp.full(idx.shape, kseq[0], dtype=jnp.int32)
    x1 = idx + kseq[1]
    for i in range(5):
        for r in _ROTS[i * 4 : i * 4 + 4]:
            x0 = x0 + x1
            x1 = _rotl(x1, r)
            x1 = x0 ^ x1
        x0 = x0 + kseq[(i + 1) % 3]
        x1 = x1 + kseq[(i + 2) % 3] + np.int32(i + 1)
    return x0 ^ x1


def _gumbel_from_bits(bits):
    fb = lax.shift_right_logical(bits, np.int32(9)) | np.int32(0x3F800000)
    fl = lax.bitcast_convert_type(fb, jnp.float32) - np.float32(1.0)
    u = fl + _TINY
    return -jnp.log(-jnp.log(u))


_NW = 32
_ELEMS = B * N
_PER_W = _ELEMS // _NW
_CHUNK = 32000
_VECS = _CHUNK // 16


_UNROLL = 4


def _sc_bits_body(out_hbm, buf0, buf1, sem0, sem1):
    wid = lax.axis_index("s") * 2 + lax.axis_index("c")
    lane = lax.iota(jnp.int32, 16)
    base = wid * np.int32(_PER_W)
    bufs, sems = (buf0, buf1), (sem0, sem1)
    pending = {}
    n_ch = _PER_W // _CHUNK
    for ch in range(n_ch):
        buf, sem = bufs[ch % 2], sems[ch % 2]
        if ch % 2 in pending:
            pending[ch % 2].wait()
        chunk_base = base + np.int32(ch * _CHUNK)

        def body(i, carry, chunk_base=chunk_base, buf=buf):
            for j in range(_UNROLL):
                idx = lane + (chunk_base + i * np.int32(16 * _UNROLL) + np.int32(16 * j))
                buf[pl.ds(i * (16 * _UNROLL) + 16 * j, 16)] = _threefry_bits(_KE, idx)
            return carry

        lax.fori_loop(0, _CHUNK // (16 * _UNROLL), body, np.int32(0))
        pending[ch % 2] = pltpu.async_copy(buf, out_hbm.at[pl.ds(chunk_base, _CHUNK)], sem)
    for h in pending.values():
        h.wait()


def _sc_bits_e():
    mesh = plsc.VectorSubcoreMesh(core_axis_name="c", subcore_axis_name="s")
    fn = pl.kernel(
        _sc_bits_body,
        mesh=mesh,
        out_type=jax.ShapeDtypeStruct((_ELEMS,), jnp.int32),
        scratch_types=[
            pltpu.VMEM((_CHUNK,), jnp.int32),
            pltpu.VMEM((_CHUNK,), jnp.int32),
            pltpu.SemaphoreType.DMA,
            pltpu.SemaphoreType.DMA,
        ],
    )
    return fn()


def _first_argmax(s, col):
    m = jnp.max(s, axis=1, keepdims=True)
    return jnp.min(jnp.where(s == m, col, np.int32(N)), axis=1)


def _body1(pot_ref, pc_ref, bt_ref, city_ref):
    i = pl.program_id(0)
    row = lax.broadcasted_iota(jnp.int32, (BB, N), 0)
    col = lax.broadcasted_iota(jnp.int32, (BB, N), 1)
    idx = (i * np.int32(BB) + row) * np.int32(N) + col

    g_b = _gumbel_from_bits(_threefry_bits(_KB, idx))
    bt_ref[...] = _first_argmax(pot_ref[...] + g_b, col)

    g_c = _gumbel_from_bits(_threefry_bits(_KC, idx))
    city_ref[...] = _first_argmax(jnp.log(pc_ref[...]) + g_c, col)


def _body2(pe_ref, tour_ref, be_ref, city_ref, es_ref, ee_ref):
    col = lax.broadcasted_iota(jnp.int32, (BB, N), 1)
    city = city_ref[...]

    g_e = _gumbel_from_bits(be_ref[...])
    s_e = jnp.where(col == city[:, None], _LOG_1E9, jnp.log(pe_ref[...])) + g_e
    ie = _first_argmax(s_e, col)
    es_ref[...] = ie

    tour = tour_ref[...]
    pos = jnp.min(jnp.where(tour == ie[:, None], col, np.int32(N)), axis=1)
    nxt = jnp.where(pos == np.int32(N - 1), np.int32(0), pos + np.int32(1))
    ee_ref[...] = jnp.sum(jnp.where(col == nxt[:, None], tour, np.int32(0)), axis=1)


def kernel(backtrack_potentials, city_to_insert_probs, edge_to_insert_probs, current_tour):
    bits_e = _sc_bits_e().reshape(B, N)
    in_spec = pl.BlockSpec((BB, N), lambda i: (i, 0))
    out_spec = pl.BlockSpec((BB,), lambda i: (i,))
    out_shape = jax.ShapeDtypeStruct((B,), jnp.int32)
    bt, city = pl.pallas_call(
        _body1,
        grid=(B // BB,),
        in_specs=[in_spec] * 2,
        out_specs=[out_spec] * 2,
        out_shape=[out_shape] * 2,
    )(backtrack_potentials, city_to_insert_probs)
    es, ee = pl.pallas_call(
        _body2,
        grid=(B // BB,),
        in_specs=[in_spec, in_spec, in_spec, out_spec],
        out_specs=[out_spec] * 2,
        out_shape=[out_shape] * 2,
    )(edge_to_insert_probs, current_tour, bits_e, city)
    return bt, city, jnp.stack([es, ee], axis=1)


# speedup vs baseline: 1.3981x; 1.0085x over previous
import functools

import numpy as np
import jax
import jax.numpy as jnp
from jax import lax
from jax.experimental import pallas as pl
from jax.experimental.pallas import tpu as pltpu
from jax.experimental.pallas import tpu_sc as plsc

B = 4096
N = 1000
BB = 256


def _np_threefry2x32(k1, k2, x0, x1):
    k1, k2 = np.uint32(k1), np.uint32(k2)
    ks = [k1, k2, np.uint32(k1 ^ k2 ^ np.uint32(0x1BD11BDA))]
    rots = [[13, 15, 26, 6], [17, 29, 16, 24]]
    x0 = (x0 + ks[0]).astype(np.uint32)
    x1 = (x1 + ks[1]).astype(np.uint32)
    for i in range(5):
        for r in rots[i % 2]:
            x0 = (x0 + x1).astype(np.uint32)
            x1 = ((x1 << np.uint32(r)) | (x1 >> np.uint32(32 - r))).astype(np.uint32)
            x1 = (x0 ^ x1).astype(np.uint32)
        x0 = (x0 + ks[(i + 1) % 3]).astype(np.uint32)
        x1 = (x1 + ks[(i + 2) % 3] + np.uint32(i + 1)).astype(np.uint32)
    return x0, x1


def _subkeys_of_42():
    idx = np.arange(3, dtype=np.uint64)
    hi = (idx >> np.uint64(32)).astype(np.uint32)
    lo = (idx & np.uint64(0xFFFFFFFF)).astype(np.uint32)
    o0, o1 = _np_threefry2x32(np.uint32(0), np.uint32(42), hi, lo)
    return [(int(o0[i]), int(o1[i])) for i in range(3)]


_KB, _KC, _KE = _subkeys_of_42()

_TINY = np.float32(np.finfo(np.float32).tiny)
_LOG_1E9 = np.float32(np.log(np.float32(1e-9)))
_ROTS = (13, 15, 26, 6, 17, 29, 16, 24, 13, 15, 26, 6, 17, 29, 16, 24, 13, 15, 26, 6)


def _as_i32(x):
    return np.array(x, dtype=np.uint32).view(np.int32)[()]


def _rotl(x, r):
    return lax.shift_left(x, np.int32(r)) | lax.shift_right_logical(x, np.int32(32 - r))


def _key_consts(key):
    k1, k2 = np.uint32(key[0]), np.uint32(key[1])
    k3 = np.uint32(k1 ^ k2 ^ np.uint32(0x1BD11BDA))
    return [_as_i32(k1), _as_i32(k2), _as_i32(k3)]


def _threefry_bits(key, idx):
    kseq = _key_consts(key)
    x0 = jnp.full(idx.shape, kseq[0], dtype=jnp.int32)
    x1 = idx + kseq[1]
    for i in range(5):
        for r in _ROTS[i * 4 : i * 4 + 4]:
            x0 = x0 + x1
            x1 = _rotl(x1, r)
            x1 = x0 ^ x1
        x0 = x0 + kseq[(i + 1) % 3]
        x1 = x1 + kseq[(i + 2) % 3] + np.int32(i + 1)
    return x0 ^ x1


def _gumbel_from_bits(bits):
    fb = lax.shift_right_logical(bits, np.int32(9)) | np.int32(0x3F800000)
    fl = lax.bitcast_convert_type(fb, jnp.float32) - np.float32(1.0)
    u = fl + _TINY
    return -jnp.log(-jnp.log(u))


_NW = 32
_NP = 1024
_ELEMS = B * _NP
_PER_W = _ELEMS // _NW
_CHUNK = 32768

_UNROLL = 4


def _sc_bits_body(out_hbm, buf0, buf1, sem0, sem1):
    wid = lax.axis_index("s") * 2 + lax.axis_index("c")
    lane = lax.iota(jnp.int32, 16)
    base = wid * np.int32(_PER_W)
    bufs, sems = (buf0, buf1), (sem0, sem1)
    pending = {}
    n_ch = _PER_W // _CHUNK
    for ch in range(n_ch):
        buf, sem = bufs[ch % 2], sems[ch % 2]
        if ch % 2 in pending:
            pending[ch % 2].wait()
        chunk_base = base + np.int32(ch * _CHUNK)

        def body(i, carry, chunk_base=chunk_base, buf=buf):
            for j in range(_UNROLL):
                o = chunk_base + i * np.int32(16 * _UNROLL) + np.int32(16 * j)
                r = lax.shift_right_logical(o, np.int32(10))
                c = o & np.int32(_NP - 1)
                idx = lane + (r * np.int32(N) + c)
                buf[pl.ds(i * (16 * _UNROLL) + 16 * j, 16)] = _threefry_bits(_KE, idx)
            return carry

        lax.fori_loop(0, _CHUNK // (16 * _UNROLL), body, np.int32(0))
        pending[ch % 2] = pltpu.async_copy(buf, out_hbm.at[pl.ds(chunk_base, _CHUNK)], sem)
    for h in pending.values():
        h.wait()


def _sc_bits_e():
    mesh = plsc.VectorSubcoreMesh(core_axis_name="c", subcore_axis_name="s")
    fn = pl.kernel(
        _sc_bits_body,
        mesh=mesh,
        out_type=jax.ShapeDtypeStruct((_ELEMS,), jnp.int32),
        scratch_types=[
            pltpu.VMEM((_CHUNK,), jnp.int32),
            pltpu.VMEM((_CHUNK,), jnp.int32),
            pltpu.SemaphoreType.DMA,
            pltpu.SemaphoreType.DMA,
        ],
    )
    return fn()


def _first_argmax(s, col):
    m = jnp.max(s, axis=1, keepdims=True)
    return jnp.min(jnp.where(s == m, col, np.int32(N)), axis=1)


def _body1(pot_ref, pc_ref, bt_ref, city_ref):
    i = pl.program_id(0)
    row = lax.broadcasted_iota(jnp.int32, (BB, N), 0)
    col = lax.broadcasted_iota(jnp.int32, (BB, N), 1)
    idx = (i * np.int32(BB) + row) * np.int32(N) + col

    g_b = _gumbel_from_bits(_threefry_bits(_KB, idx))
    bt_ref[...] = _first_argmax(pot_ref[...] + g_b, col)

    g_c = _gumbel_from_bits(_threefry_bits(_KC, idx))
    city_ref[...] = _first_argmax(jnp.log(pc_ref[...]) + g_c, col)


def _body2(pe_ref, tour_ref, be_ref, city_ref, es_ref, ee_ref):
    col = lax.broadcasted_iota(jnp.int32, (BB, N), 1)
    city = city_ref[...]

    g_e = _gumbel_from_bits(be_ref[:, : N])
    s_e = jnp.where(col == city[:, None], _LOG_1E9, jnp.log(pe_ref[...])) + g_e
    ie = _first_argmax(s_e, col)
    es_ref[...] = ie

    tour = tour_ref[...]
    pos = jnp.min(jnp.where(tour == ie[:, None], col, np.int32(N)), axis=1)
    nxt = jnp.where(pos == np.int32(N - 1), np.int32(0), pos + np.int32(1))
    ee_ref[...] = jnp.sum(jnp.where(col == nxt[:, None], tour, np.int32(0)), axis=1)


def kernel(backtrack_potentials, city_to_insert_probs, edge_to_insert_probs, current_tour):
    bits_e = _sc_bits_e().reshape(B, _NP)
    in_spec = pl.BlockSpec((BB, N), lambda i: (i, 0))
    bits_spec = pl.BlockSpec((BB, _NP), lambda i: (i, 0))
    out_spec = pl.BlockSpec((BB,), lambda i: (i,))
    out_shape = jax.ShapeDtypeStruct((B,), jnp.int32)
    bt, city = pl.pallas_call(
        _body1,
        grid=(B // BB,),
        in_specs=[in_spec] * 2,
        out_specs=[out_spec] * 2,
        out_shape=[out_shape] * 2,
    )(backtrack_potentials, city_to_insert_probs)
    es, ee = pl.pallas_call(
        _body2,
        grid=(B // BB,),
        in_specs=[in_spec, in_spec, bits_spec, out_spec],
        out_specs=[out_spec] * 2,
        out_shape=[out_shape] * 2,
    )(edge_to_insert_probs, current_tour, bits_e, city)
    return bt, city, jnp.stack([es, ee], axis=1)


# speedup vs baseline: 1.4931x; 1.0679x over previous
import functools

import numpy as np
import jax
import jax.numpy as jnp
from jax import lax
from jax.experimental import pallas as pl
from jax.experimental.pallas import tpu as pltpu
from jax.experimental.pallas import tpu_sc as plsc

B = 4096
N = 1000
BB = 256


def _np_threefry2x32(k1, k2, x0, x1):
    k1, k2 = np.uint32(k1), np.uint32(k2)
    ks = [k1, k2, np.uint32(k1 ^ k2 ^ np.uint32(0x1BD11BDA))]
    rots = [[13, 15, 26, 6], [17, 29, 16, 24]]
    x0 = (x0 + ks[0]).astype(np.uint32)
    x1 = (x1 + ks[1]).astype(np.uint32)
    for i in range(5):
        for r in rots[i % 2]:
            x0 = (x0 + x1).astype(np.uint32)
            x1 = ((x1 << np.uint32(r)) | (x1 >> np.uint32(32 - r))).astype(np.uint32)
            x1 = (x0 ^ x1).astype(np.uint32)
        x0 = (x0 + ks[(i + 1) % 3]).astype(np.uint32)
        x1 = (x1 + ks[(i + 2) % 3] + np.uint32(i + 1)).astype(np.uint32)
    return x0, x1


def _subkeys_of_42():
    idx = np.arange(3, dtype=np.uint64)
    hi = (idx >> np.uint64(32)).astype(np.uint32)
    lo = (idx & np.uint64(0xFFFFFFFF)).astype(np.uint32)
    o0, o1 = _np_threefry2x32(np.uint32(0), np.uint32(42), hi, lo)
    return [(int(o0[i]), int(o1[i])) for i in range(3)]


_KB, _KC, _KE = _subkeys_of_42()

_TINY = np.float32(np.finfo(np.float32).tiny)
_LOG_1E9 = np.float32(np.log(np.float32(1e-9)))
_ROTS = (13, 15, 26, 6, 17, 29, 16, 24, 13, 15, 26, 6, 17, 29, 16, 24, 13, 15, 26, 6)


def _as_i32(x):
    return np.array(x, dtype=np.uint32).view(np.int32)[()]


def _rotl(x, r):
    return lax.shift_left(x, np.int32(r)) | lax.shift_right_logical(x, np.int32(32 - r))


def _key_consts(key):
    k1, k2 = np.uint32(key[0]), np.uint32(key[1])
    k3 = np.uint32(k1 ^ k2 ^ np.uint32(0x1BD11BDA))
    return [_as_i32(k1), _as_i32(k2), _as_i32(k3)]


def _threefry_bits(key, idx):
    kseq = _key_consts(key)
    x0 = jnp.full(idx.shape, kseq[0], dtype=jnp.int32)
    x1 = idx + kseq[1]
    for i in range(5):
        for r in _ROTS[i * 4 : i * 4 + 4]:
            x0 = x0 + x1
            x1 = _rotl(x1, r)
            x1 = x0 ^ x1
        x0 = x0 + kseq[(i + 1) % 3]
        x1 = x1 + kseq[(i + 2) % 3] + np.int32(i + 1)
    return x0 ^ x1


def _gumbel_from_bits(bits):
    fb = lax.shift_right_logical(bits, np.int32(9)) | np.int32(0x3F800000)
    fl = lax.bitcast_convert_type(fb, jnp.float32) - np.float32(1.0)
    u = fl + _TINY
    return -jnp.log(-jnp.log(u))


_NW = 32
_NP = 1024
_ELEMS = B * _NP
_PER_W = _ELEMS // _NW
_CHUNK = 32768

_UNROLL = 4


_CROWS = _CHUNK // _NP


def _sc_bits_body(out_hbm, buf0, buf1, sem0, sem1):
    wid = lax.axis_index("s") * 2 + lax.axis_index("c")
    lane = lax.iota(jnp.int32, 16)
    row_base = wid * np.int32(B // _NW)
    bufs, sems = (buf0, buf1), (sem0, sem1)
    pending = {}
    n_ch = (B // _NW) // _CROWS
    for ch in range(n_ch):
        buf, sem = bufs[ch % 2], sems[ch % 2]
        if ch % 2 in pending:
            pending[ch % 2].wait()
        row0 = row_base + np.int32(ch * _CROWS)

        def body(i, carry, row0=row0, buf=buf):
            for j in range(_UNROLL):
                v = i * np.int32(_UNROLL) + np.int32(j)
                rr = lax.shift_right_logical(v, np.int32(6))
                cc = pl.multiple_of(lax.shift_left(v & np.int32(63), np.int32(4)), 16)
                idx = lane + ((row0 + rr) * np.int32(N) + cc)
                buf[rr, pl.ds(cc, 16)] = _threefry_bits(_KE, idx)
            return carry

        lax.fori_loop(0, _CHUNK // (16 * _UNROLL), body, np.int32(0))
        pending[ch % 2] = pltpu.async_copy(buf, out_hbm.at[pl.ds(row0, _CROWS), :], sem)
    for h in pending.values():
        h.wait()


def _sc_bits_e():
    mesh = plsc.VectorSubcoreMesh(core_axis_name="c", subcore_axis_name="s")
    fn = pl.kernel(
        _sc_bits_body,
        mesh=mesh,
        out_type=jax.ShapeDtypeStruct((B, _NP), jnp.int32),
        scratch_types=[
            pltpu.VMEM((_CROWS, _NP), jnp.int32),
            pltpu.VMEM((_CROWS, _NP), jnp.int32),
            pltpu.SemaphoreType.DMA,
            pltpu.SemaphoreType.DMA,
        ],
    )
    return fn()


def _first_argmax(s, col):
    m = jnp.max(s, axis=1, keepdims=True)
    return jnp.min(jnp.where(s == m, col, np.int32(N)), axis=1)


def _body1(pot_ref, pc_ref, bt_ref, city_ref):
    i = pl.program_id(0)
    row = lax.broadcasted_iota(jnp.int32, (BB, N), 0)
    col = lax.broadcasted_iota(jnp.int32, (BB, N), 1)
    idx = (i * np.int32(BB) + row) * np.int32(N) + col

    g_b = _gumbel_from_bits(_threefry_bits(_KB, idx))
    bt_ref[...] = _first_argmax(pot_ref[...] + g_b, col)

    g_c = _gumbel_from_bits(_threefry_bits(_KC, idx))
    city_ref[...] = _first_argmax(jnp.log(pc_ref[...]) + g_c, col)


def _body2(pe_ref, tour_ref, be_ref, city_ref, es_ref, ee_ref):
    col = lax.broadcasted_iota(jnp.int32, (BB, N), 1)
    city = city_ref[...]

    g_e = _gumbel_from_bits(be_ref[:, : N])
    s_e = jnp.where(col == city[:, None], _LOG_1E9, jnp.log(pe_ref[...])) + g_e
    ie = _first_argmax(s_e, col)
    es_ref[...] = ie

    tour = tour_ref[...]
    pos = jnp.min(jnp.where(tour == ie[:, None], col, np.int32(N)), axis=1)
    nxt = jnp.where(pos == np.int32(N - 1), np.int32(0), pos + np.int32(1))
    ee_ref[...] = jnp.sum(jnp.where(col == nxt[:, None], tour, np.int32(0)), axis=1)


def kernel(backtrack_potentials, city_to_insert_probs, edge_to_insert_probs, current_tour):
    bits_e = _sc_bits_e()
    in_spec = pl.BlockSpec((BB, N), lambda i: (i, 0))
    bits_spec = pl.BlockSpec((BB, _NP), lambda i: (i, 0))
    out_spec = pl.BlockSpec((BB,), lambda i: (i,))
    out_shape = jax.ShapeDtypeStruct((B,), jnp.int32)
    bt, city = pl.pallas_call(
        _body1,
        grid=(B // BB,),
        in_specs=[in_spec] * 2,
        out_specs=[out_spec] * 2,
        out_shape=[out_shape] * 2,
    )(backtrack_potentials, city_to_insert_probs)
    es, ee = pl.pallas_call(
        _body2,
        grid=(B // BB,),
        in_specs=[in_spec, in_spec, bits_spec, out_spec],
        out_specs=[out_spec] * 2,
        out_shape=[out_shape] * 2,
    )(edge_to_insert_probs, current_tour, bits_e, city)
    return bt, city, jnp.stack([es, ee], axis=1)
